# baseline (speedup 1.0000x reference)
import jax
import jax.numpy as jnp
from jax import lax
from jax.experimental import pallas as pl
from jax.experimental.pallas import tpu as pltpu
from jax.experimental.pallas import tpu_sc as plsc

B, N, D = 4096, 128, 64
ROW = N * D
NC, NS = 2, 16
NW = NC * NS
BPW = B // NW
CH = 8
NCHUNK = BPW // CH
CHW = CH * ROW
WEIGHT = 0.05


def _sc_body(x_hbm, out_hbm, buf, acc_v):
    wid = lax.axis_index("s") * NC + lax.axis_index("c")
    base = wid * (BPW * ROW)
    zero = jnp.zeros((16,), jnp.float32)

    def chunk_loop(g, accs):
        pltpu.sync_copy(x_hbm.at[pl.ds(base + g * CHW, CHW)], buf)

        def b_loop(b, accs):
            b0 = b * ROW
            r0 = buf[pl.ds(b0, 16)]
            r1 = buf[pl.ds(b0 + 16, 16)]
            r2 = buf[pl.ds(b0 + 32, 16)]
            r3 = buf[pl.ds(b0 + 48, 16)]

            def row_loop(i, carry):
                p0, p1, p2, p3, a0, a1, a2, a3 = carry
                o = b0 + i * D
                s0 = buf[pl.ds(o, 16)]
                s1 = buf[pl.ds(o + 16, 16)]
                s2 = buf[pl.ds(o + 32, 16)]
                s3 = buf[pl.ds(o + 48, 16)]
                d0 = p0 - s0
                d1 = p1 - s1
                d2 = p2 - s2
                d3 = p3 - s3
                return (s0, s1, s2, s3,
                        a0 + d0 * d0, a1 + d1 * d1,
                        a2 + d2 * d2, a3 + d3 * d3)

            out = lax.fori_loop(1, N, row_loop, (r0, r1, r2, r3) + accs)
            return out[4:]

        return lax.fori_loop(0, CH, b_loop, accs)

    a0, a1, a2, a3 = lax.fori_loop(0, NCHUNK, chunk_loop,
                                   (zero, zero, zero, zero))
    acc_v[...] = (a0 + a1) + (a2 + a3)
    pltpu.sync_copy(acc_v, out_hbm.at[wid])


@jax.jit
def kernel(coords):
    x = coords.reshape(B * ROW)
    mesh = plsc.VectorSubcoreMesh(core_axis_name="c", subcore_axis_name="s",
                                  num_cores=NC, num_subcores=NS)
    partials = pl.kernel(
        _sc_body,
        out_type=jax.ShapeDtypeStruct((NW, 16), jnp.float32),
        mesh=mesh,
        scratch_types=[
            pltpu.VMEM((CHW,), jnp.float32),
            pltpu.VMEM((16,), jnp.float32),
        ],
    )(x)
    return (WEIGHT / (B * (N - 1))) * jnp.sum(partials)


# speedup vs baseline: 1.1444x; 1.1444x over previous
import jax
import jax.numpy as jnp
from jax import lax
from jax.experimental import pallas as pl
from jax.experimental.pallas import tpu as pltpu
from jax.experimental.pallas import tpu_sc as plsc

B, N, D = 4096, 128, 64
ROW = N * D
NC, NS = 2, 16
NW = NC * NS
BPW = B // NW
CH = 4
NCHUNK = BPW // CH
CHW = CH * ROW
NVEC = (CHW - D) // 16
NACC = 14
NNEG = 2
WEIGHT = 0.05


def _sc_body(x_hbm, out_hbm, buf0, buf1, acc_v, sem0, sem1):
    wid = lax.axis_index("s") * NC + lax.axis_index("c")
    base = wid * (BPW * ROW)
    zero = jnp.zeros((16,), jnp.float32)

    def start(c, buf, sem):
        pltpu.async_copy(x_hbm.at[pl.ds(base + c * CHW, CHW)], buf, sem)

    def wait(buf, sem):
        pltpu.make_async_copy(x_hbm.at[pl.ds(base, CHW)], buf, sem).wait()

    def compute(buf, accs, negs):
        @plsc.parallel_loop(0, CHW - D, 16 * NACC, unroll=2, carry=accs)
        def accs(k, accs):
            new = []
            for j in range(NACC):
                v1 = buf[pl.ds(k + j * 16, 16)]
                v2 = buf[pl.ds(k + j * 16 + D, 16)]
                d = v1 - v2
                new.append(accs[j] + d * d)
            return tuple(new)

        n0, n1 = negs
        for bb in range(CH - 1):
            o = bb * ROW + (N - 1) * D
            for j in range(D // 16):
                v1 = buf[pl.ds(o + j * 16, 16)]
                v2 = buf[pl.ds(o + j * 16 + D, 16)]
                d = v1 - v2
                if j % 2 == 0:
                    n0 = n0 + d * d
                else:
                    n1 = n1 + d * d
        return accs, (n0, n1)

    start(0, buf0, sem0)

    def outer(g, carry):
        accs, negs = carry[:NACC], carry[NACC:]
        c0 = 2 * g
        wait(buf0, sem0)
        start(c0 + 1, buf1, sem1)
        accs, negs = compute(buf0, accs, negs)
        wait(buf1, sem1)

        @pl.when(c0 + 2 < NCHUNK)
        def _():
            start(c0 + 2, buf0, sem0)

        accs, negs = compute(buf1, accs, negs)
        return accs + negs

    carry = lax.fori_loop(0, NCHUNK // 2, outer, (zero,) * (NACC + NNEG))
    tot = carry[0]
    for a in carry[1:NACC]:
        tot = tot + a
    for ng in carry[NACC:]:
        tot = tot - ng
    acc_v[...] = tot
    pltpu.sync_copy(acc_v, out_hbm.at[wid])


@jax.jit
def kernel(coords):
    x = coords.reshape(B * ROW)
    mesh = plsc.VectorSubcoreMesh(core_axis_name="c", subcore_axis_name="s",
                                  num_cores=NC, num_subcores=NS)
    partials = pl.kernel(
        _sc_body,
        out_type=jax.ShapeDtypeStruct((NW, 16), jnp.float32),
        mesh=mesh,
        scratch_types=[
            pltpu.VMEM((CHW,), jnp.float32),
            pltpu.VMEM((CHW,), jnp.float32),
            pltpu.VMEM((16,), jnp.float32),
            pltpu.SemaphoreType.DMA,
            pltpu.SemaphoreType.DMA,
        ],
    )(x)
    return (WEIGHT / (B * (N - 1))) * jnp.sum(partials)


# speedup vs baseline: 1.3602x; 1.1886x over previous
import jax
import jax.numpy as jnp
from jax import lax
from jax.experimental import pallas as pl
from jax.experimental.pallas import tpu as pltpu
from jax.experimental.pallas import tpu_sc as plsc

B, N, D = 4096, 128, 64
NC, NS = 2, 16
NW = NC * NS
BPW = B // NW
CH = 2
NCHUNK = BPW // CH
NACC = 4
WEIGHT = 0.05


def _sc_body(x_hbm, out_hbm, buf0, buf1, acc_v, sem0, sem1):
    wid = lax.axis_index("s") * NC + lax.axis_index("c")
    base = wid * BPW
    zero = jnp.zeros((16,), jnp.float32)

    def start(c, buf, sem):
        pltpu.async_copy(x_hbm.at[pl.ds(base + c * CH, CH)], buf, sem)

    def wait(buf, sem):
        pltpu.make_async_copy(x_hbm.at[pl.ds(base, CH)], buf, sem).wait()

    def compute(buf, accs):
        for bb in range(CH):
            @plsc.parallel_loop(0, N - 2, 1, unroll=7, carry=accs)
            def accs(n, accs):
                new = []
                for j in range(NACC):
                    v1 = buf[bb, n, pl.ds(j * 16, 16)]
                    v2 = buf[bb, n + 1, pl.ds(j * 16, 16)]
                    d = v1 - v2
                    new.append(accs[j] + d * d)
                return tuple(new)

            new = []
            for j in range(NACC):
                v1 = buf[bb, N - 2, pl.ds(j * 16, 16)]
                v2 = buf[bb, N - 1, pl.ds(j * 16, 16)]
                d = v1 - v2
                new.append(accs[j] + d * d)
            accs = tuple(new)
        return accs

    start(0, buf0, sem0)

    def outer(g, accs):
        c0 = 2 * g
        wait(buf0, sem0)
        start(c0 + 1, buf1, sem1)
        accs = compute(buf0, accs)
        wait(buf1, sem1)

        @pl.when(c0 + 2 < NCHUNK)
        def _():
            start(c0 + 2, buf0, sem0)

        return compute(buf1, accs)

    accs = lax.fori_loop(0, NCHUNK // 2, outer, (zero,) * NACC)
    acc_v[...] = (accs[0] + accs[1]) + (accs[2] + accs[3])
    pltpu.sync_copy(acc_v, out_hbm.at[wid])


@jax.jit
def kernel(coords):
    mesh = plsc.VectorSubcoreMesh(core_axis_name="c", subcore_axis_name="s",
                                  num_cores=NC, num_subcores=NS)
    partials = pl.kernel(
        _sc_body,
        out_type=jax.ShapeDtypeStruct((NW, 16), jnp.float32),
        mesh=mesh,
        scratch_types=[
            pltpu.VMEM((CH, N, D), jnp.float32),
            pltpu.VMEM((CH, N, D), jnp.float32),
            pltpu.VMEM((16,), jnp.float32),
            pltpu.SemaphoreType.DMA,
            pltpu.SemaphoreType.DMA,
        ],
    )(coords)
    return (WEIGHT / (B * (N - 1))) * jnp.sum(partials)


# speedup vs baseline: 4.1577x; 3.0566x over previous
import jax
import jax.numpy as jnp
from jax import lax
from jax.experimental import pallas as pl
from jax.experimental.pallas import tpu as pltpu
from jax.experimental.pallas import tpu_sc as plsc

B, N, D = 4096, 128, 64
ROW = N * D
NC, NS = 2, 16
NW = NC * NS
BPW = B // NW
CH = 2
NCHUNK = BPW // CH
CHW = CH * ROW
WEIGHT = 0.05


def _sc_body(x_hbm, out_hbm, buf0, buf1, acc_v, sem0, sem1):
    wid = lax.axis_index("s") * NC + lax.axis_index("c")
    base = wid * (BPW * ROW)
    zero = jnp.zeros((16,), jnp.float32)
    mask = jnp.where(lax.iota(jnp.int32, 16) < 15, 1.0, 0.0).astype(jnp.float32)

    buf0[pl.ds(CHW, 16)] = zero
    buf1[pl.ds(CHW, 16)] = zero

    def start(c, buf, sem):
        pltpu.async_copy(x_hbm.at[pl.ds(base + c * CHW, CHW)],
                         buf.at[pl.ds(0, CHW)], sem)

    def wait(buf, sem):
        pltpu.make_async_copy(x_hbm.at[pl.ds(base, CHW)],
                              buf.at[pl.ds(0, CHW)], sem).wait()

    def compute(buf, accs):
        @plsc.parallel_loop(0, CHW, N, unroll=4, carry=accs)
        def accs(o, accs):
            new = list(accs)
            for j in range(8):
                va = buf[pl.ds(o + 16 * j, 16)]
                vs = buf[pl.ds(o + 16 * j + 1, 16)]
                d = va - vs
                if j == 7:
                    d = d * mask
                new[j] = new[j] + d * d
            return tuple(new)
        return accs

    start(0, buf0, sem0)

    def outer(g, accs):
        c0 = 2 * g
        wait(buf0, sem0)
        start(c0 + 1, buf1, sem1)
        accs = compute(buf0, accs)
        wait(buf1, sem1)

        @pl.when(c0 + 2 < NCHUNK)
        def _():
            start(c0 + 2, buf0, sem0)

        return compute(buf1, accs)

    accs = lax.fori_loop(0, NCHUNK // 2, outer, (zero,) * 8)
    acc_v[...] = ((accs[0] + accs[1]) + (accs[2] + accs[3])) + \
                 ((accs[4] + accs[5]) + (accs[6] + accs[7]))
    pltpu.sync_copy(acc_v, out_hbm.at[wid])


@jax.jit
def kernel(coords):
    x = coords.transpose(0, 2, 1).reshape(B * ROW)
    mesh = plsc.VectorSubcoreMesh(core_axis_name="c", subcore_axis_name="s",
                                  num_cores=NC, num_subcores=NS)
    partials = pl.kernel(
        _sc_body,
        out_type=jax.ShapeDtypeStruct((NW, 16), jnp.float32),
        mesh=mesh,
        scratch_types=[
            pltpu.VMEM((CHW + 16,), jnp.float32),
            pltpu.VMEM((CHW + 16,), jnp.float32),
            pltpu.VMEM((16,), jnp.float32),
            pltpu.SemaphoreType.DMA,
            pltpu.SemaphoreType.DMA,
        ],
    )(x)
    return (WEIGHT / (B * (N - 1))) * jnp.sum(partials)
